# Initial kernel scaffold; baseline (speedup 1.0000x reference)
import jax
import jax.numpy as jnp
from jax.experimental import pallas as pl


def kernel(y_pred, y_true, basin, weights):
    raise NotImplementedError("write your pallas kernel here")


# speedup vs baseline: 323.6756x; 323.6756x over previous
import functools

import jax
import jax.numpy as jnp
from jax import lax
from jax.experimental import pallas as pl
from jax.experimental.pallas import tpu as pltpu
from jax.experimental.pallas import tpu_sc as plsc

N = 3276800
NUM_BASINS_PAD = 1024
NC = 2
NS = 16
L = 16
NW = NC * NS
PER_W = N // NW
CHUNK = 25600
NCHUNK = PER_W // CHUNK

_mesh = plsc.VectorSubcoreMesh(
    core_axis_name="c", subcore_axis_name="s", num_cores=NC, num_subcores=NS
)


@functools.partial(
    pl.kernel,
    out_type=jax.ShapeDtypeStruct((NW, L), jnp.float32),
    mesh=_mesh,
    scratch_types=[
        pltpu.VMEM((NUM_BASINS_PAD,), jnp.float32),
        pltpu.VMEM((CHUNK,), jnp.float32),
        pltpu.VMEM((CHUNK,), jnp.float32),
        pltpu.VMEM((CHUNK,), jnp.int32),
        pltpu.VMEM((L,), jnp.float32),
    ],
    compiler_params=pltpu.CompilerParams(needs_layout_passes=False),
)
def _nmse_partials(y_pred, y_true, basin, weights, out, w_v, p_v, t_v, b_v, o_v):
    wid = lax.axis_index("s") * NC + lax.axis_index("c")
    base = wid * PER_W
    pltpu.sync_copy(weights, w_v)

    def chunk_loop(g, acc):
        off = base + g * CHUNK
        pltpu.sync_copy(y_pred.at[pl.ds(off, CHUNK)], p_v)
        pltpu.sync_copy(y_true.at[pl.ds(off, CHUNK)], t_v)
        pltpu.sync_copy(basin.at[pl.ds(off, CHUNK)], b_v)

        def body(i, acc):
            s = pl.ds(i * L, L)
            idx = b_v[s]
            p = p_v[s]
            t = t_v[s]
            w = plsc.load_gather(w_v, [idx])
            d = p - t
            return acc + w * (d * d)

        return lax.fori_loop(0, CHUNK // L, body, acc)

    acc = lax.fori_loop(0, NCHUNK, chunk_loop, jnp.zeros((L,), jnp.float32))
    o_v[...] = acc
    pltpu.sync_copy(o_v, out.at[wid])


def kernel(y_pred, y_true, basin, weights):
    wpad = jnp.concatenate(
        [weights, jnp.zeros((NUM_BASINS_PAD - weights.shape[0],), weights.dtype)]
    )
    partials = _nmse_partials(y_pred, y_true, basin.astype(jnp.int32), wpad)
    return jnp.sum(partials) / jnp.float32(N)


# speedup vs baseline: 361.2798x; 1.1162x over previous
import functools

import jax
import jax.numpy as jnp
from jax import lax
from jax.experimental import pallas as pl
from jax.experimental.pallas import tpu as pltpu
from jax.experimental.pallas import tpu_sc as plsc

N = 3276800
NUM_BASINS_PAD = 1024
NC = 2
NS = 16
L = 16
NW = NC * NS
PER_W = N // NW
CHUNK = 12800
NCHUNK = PER_W // CHUNK

_mesh = plsc.VectorSubcoreMesh(
    core_axis_name="c", subcore_axis_name="s", num_cores=NC, num_subcores=NS
)


@functools.partial(
    pl.kernel,
    out_type=jax.ShapeDtypeStruct((NW, L), jnp.float32),
    mesh=_mesh,
    scratch_types=[
        pltpu.VMEM((NUM_BASINS_PAD,), jnp.float32),
        pltpu.VMEM((2, CHUNK), jnp.float32),
        pltpu.VMEM((2, CHUNK), jnp.float32),
        pltpu.VMEM((2, CHUNK), jnp.int32),
        pltpu.VMEM((L,), jnp.float32),
        pltpu.SemaphoreType.DMA,
        pltpu.SemaphoreType.DMA,
    ],
    compiler_params=pltpu.CompilerParams(needs_layout_passes=False),
)
def _nmse_partials(
    y_pred, y_true, basin, weights, out, w_v, p_v, t_v, b_v, o_v, sem0, sem1
):
    wid = lax.axis_index("s") * NC + lax.axis_index("c")
    base = wid * PER_W
    pltpu.sync_copy(weights, w_v)
    sems = (sem0, sem1)

    def start(slot, g):
        off = base + g * CHUNK
        pltpu.async_copy(y_pred.at[pl.ds(off, CHUNK)], p_v.at[slot], sems[slot])
        pltpu.async_copy(y_true.at[pl.ds(off, CHUNK)], t_v.at[slot], sems[slot])
        pltpu.async_copy(basin.at[pl.ds(off, CHUNK)], b_v.at[slot], sems[slot])

    def wait(slot, g):
        off = base + g * CHUNK
        pltpu.make_async_copy(y_pred.at[pl.ds(off, CHUNK)], p_v.at[slot], sems[slot]).wait()
        pltpu.make_async_copy(y_true.at[pl.ds(off, CHUNK)], t_v.at[slot], sems[slot]).wait()
        pltpu.make_async_copy(basin.at[pl.ds(off, CHUNK)], b_v.at[slot], sems[slot]).wait()

    def compute(slot, acc):
        def body(i, acc):
            s = pl.ds(i * L, L)
            idx = b_v[slot, s]
            p = p_v[slot, s]
            t = t_v[slot, s]
            w = plsc.load_gather(w_v, [idx])
            d = p - t
            return acc + w * (d * d)

        return lax.fori_loop(0, CHUNK // L, body, acc)

    start(0, 0)

    def step(s, acc):
        g0 = 2 * s
        start(1, g0 + 1)
        wait(0, g0)
        acc = compute(0, acc)

        @pl.when(g0 + 2 < NCHUNK)
        def _():
            start(0, g0 + 2)

        wait(1, g0 + 1)
        return compute(1, acc)

    acc = lax.fori_loop(0, NCHUNK // 2, step, jnp.zeros((L,), jnp.float32))
    o_v[...] = acc
    pltpu.sync_copy(o_v, out.at[wid])


def kernel(y_pred, y_true, basin, weights):
    wpad = jnp.concatenate(
        [weights, jnp.zeros((NUM_BASINS_PAD - weights.shape[0],), weights.dtype)]
    )
    partials = _nmse_partials(y_pred, y_true, basin.astype(jnp.int32), wpad)
    return jnp.sum(partials) / jnp.float32(N)


# speedup vs baseline: 507.8179x; 1.4056x over previous
import functools

import jax
import jax.numpy as jnp
from jax import lax
from jax.experimental import pallas as pl
from jax.experimental.pallas import tpu as pltpu
from jax.experimental.pallas import tpu_sc as plsc

N = 3276800
NUM_BASINS_PAD = 1024
NC = 2
NS = 16
L = 16
NW = NC * NS
PER_W = N // NW
CHUNK = 12800
NCHUNK = PER_W // CHUNK

_mesh = plsc.VectorSubcoreMesh(
    core_axis_name="c", subcore_axis_name="s", num_cores=NC, num_subcores=NS
)


@functools.partial(
    pl.kernel,
    out_type=jax.ShapeDtypeStruct((NW, L), jnp.float32),
    mesh=_mesh,
    scratch_types=[
        pltpu.VMEM((NUM_BASINS_PAD,), jnp.float32),
        pltpu.VMEM((2, CHUNK), jnp.float32),
        pltpu.VMEM((2, CHUNK), jnp.float32),
        pltpu.VMEM((2, CHUNK), jnp.int32),
        pltpu.VMEM((L,), jnp.float32),
        pltpu.SemaphoreType.DMA,
        pltpu.SemaphoreType.DMA,
    ],
    compiler_params=pltpu.CompilerParams(needs_layout_passes=False),
)
def _nmse_partials(
    y_pred, y_true, basin, weights, out, w_v, p_v, t_v, b_v, o_v, sem0, sem1
):
    wid = lax.axis_index("s") * NC + lax.axis_index("c")
    base = wid * PER_W
    pltpu.sync_copy(weights, w_v)
    sems = (sem0, sem1)

    def start(slot, g):
        off = base + g * CHUNK
        pltpu.async_copy(y_pred.at[pl.ds(off, CHUNK)], p_v.at[slot], sems[slot])
        pltpu.async_copy(y_true.at[pl.ds(off, CHUNK)], t_v.at[slot], sems[slot])
        pltpu.async_copy(basin.at[pl.ds(off, CHUNK)], b_v.at[slot], sems[slot])

    def wait(slot, g):
        off = base + g * CHUNK
        pltpu.make_async_copy(y_pred.at[pl.ds(off, CHUNK)], p_v.at[slot], sems[slot]).wait()
        pltpu.make_async_copy(y_true.at[pl.ds(off, CHUNK)], t_v.at[slot], sems[slot]).wait()
        pltpu.make_async_copy(basin.at[pl.ds(off, CHUNK)], b_v.at[slot], sems[slot]).wait()

    def compute(slot, acc):
        @plsc.parallel_loop(
            0, CHUNK, step=4 * L, unroll=2,
            carry=(acc, jnp.zeros((L,), jnp.float32),
                   jnp.zeros((L,), jnp.float32), jnp.zeros((L,), jnp.float32)),
        )
        def accs(i, accs):
            out = []
            for k in range(4):
                s = pl.ds(i + k * L, L)
                idx = b_v[slot, s]
                p = p_v[slot, s]
                t = t_v[slot, s]
                w = plsc.load_gather(w_v, [idx])
                d = p - t
                out.append(accs[k] + w * (d * d))
            return tuple(out)

        return (accs[0] + accs[1]) + (accs[2] + accs[3])

    start(0, 0)

    def step(s, acc):
        g0 = 2 * s
        start(1, g0 + 1)
        wait(0, g0)
        acc = compute(0, acc)

        @pl.when(g0 + 2 < NCHUNK)
        def _():
            start(0, g0 + 2)

        wait(1, g0 + 1)
        return compute(1, acc)

    acc = lax.fori_loop(0, NCHUNK // 2, step, jnp.zeros((L,), jnp.float32))
    o_v[...] = acc
    pltpu.sync_copy(o_v, out.at[wid])


def kernel(y_pred, y_true, basin, weights):
    wpad = jnp.concatenate(
        [weights, jnp.zeros((NUM_BASINS_PAD - weights.shape[0],), weights.dtype)]
    )
    partials = _nmse_partials(y_pred, y_true, basin.astype(jnp.int32), wpad)
    return jnp.sum(partials) / jnp.float32(N)
